# baseline (speedup 1.0000x reference)
import functools

import jax
import jax.numpy as jnp
from jax import lax
from jax.experimental import pallas as pl
from jax.experimental.pallas import tpu as pltpu
from jax.experimental.pallas import tpu_sc as plsc

EMBEDDING_DIM = 64


@functools.cache
def _make_flatten(T: int, B0: int, t0: int, th: int):
    n_workers = 32
    bw = B0 // n_workers
    mesh = plsc.VectorSubcoreMesh(core_axis_name="c", subcore_axis_name="s")

    @functools.partial(
        pl.kernel,
        mesh=mesh,
        compiler_params=pltpu.CompilerParams(needs_layout_passes=False),
        out_type=jax.ShapeDtypeStruct((th * B0,), jnp.int32),
        scratch_types=[
            pltpu.VMEM((th, bw), jnp.int32),
            pltpu.VMEM((th * bw,), jnp.int32),
        ],
    )
    def k(xt_hbm, flat_hbm, inb, outb):
        wid = lax.axis_index("s") * 2 + lax.axis_index("c")
        col0 = wid * bw
        pltpu.sync_copy(xt_hbm.at[pl.ds(t0, th), pl.ds(col0, bw)], inb)

        iota = lax.iota(jnp.int32, 16)

        def tbody(t, carry):
            for g in range(bw // 16):
                v = inb[t, pl.ds(g * 16, 16)]
                addr = (iota + g * 16) * th + t
                plsc.store_scatter(outb, [addr], v)
            return carry

        lax.fori_loop(0, th, tbody, 0)
        pltpu.sync_copy(outb, flat_hbm.at[pl.ds(col0 * th, bw * th)])

    return k


@functools.cache
def _make_sc_gather(B: int, D: int, n_workers: int, chunk: int, nbuf: int):
    b_per_w = B // n_workers
    n_chunks = b_per_w // chunk
    n_rounds = (n_chunks + nbuf - 1) // nbuf
    mesh = plsc.VectorSubcoreMesh(core_axis_name="c", subcore_axis_name="s")

    @functools.partial(
        pl.kernel,
        mesh=mesh,
        compiler_params=pltpu.CompilerParams(use_tc_tiling_on_sc=False),
        out_type=jax.ShapeDtypeStruct((B, D), jnp.float32),
        scratch_types=[
            pltpu.VMEM((b_per_w,), jnp.int32),
            pltpu.VMEM((nbuf, chunk, D), jnp.float32),
            pltpu.SemaphoreType.DMA((nbuf,)),
            pltpu.SemaphoreType.DMA((nbuf,)),
        ],
    )
    def k(table_hbm, idx_hbm, out_hbm, idx_v, rows_v, gsem, osem):
        wid = lax.axis_index("s") * 2 + lax.axis_index("c")
        base0 = wid * b_per_w
        pltpu.sync_copy(idx_hbm.at[pl.ds(base0, b_per_w)], idx_v)

        def gather(i, b):
            off = pl.multiple_of(i * chunk, chunk)
            return pltpu.make_async_copy(
                table_hbm.at[idx_v.at[pl.ds(off, chunk)]], rows_v.at[b],
                gsem.at[b]
            )

        def write(i, b):
            off = pl.multiple_of(base0 + i * chunk, chunk)
            return pltpu.make_async_copy(
                rows_v.at[b], out_hbm.at[pl.ds(off, chunk)], osem.at[b]
            )

        for b in range(nbuf):
            gather(b, b).start()

        def round_body(r, carry):
            for b in range(nbuf):
                i = r * nbuf + b

                @pl.when(i < n_chunks)
                def _():
                    gather(i, b).wait()
                    write(i, b).start()
                    nxt = i + nbuf

                    @pl.when(nxt < n_chunks)
                    def _():
                        write(i, b).wait()
                        gather(nxt, b).start()

            return carry

        lax.fori_loop(0, n_rounds, round_body, 0)

        for b in range(nbuf):
            last_i = ((n_chunks - 1 - b) // nbuf) * nbuf + b
            write(last_i, b).wait()

    return k


def kernel(x, W):
    B0, T = x.shape
    xt = x.T.astype(jnp.int32)
    halves = []
    for t0, th in ((0, 24), (24, T - 24)):
        flat_idx = _make_flatten(T, B0, t0, th)(xt)
        gather = _make_sc_gather(B0 * th, EMBEDDING_DIM, 32, 512, 3)
        halves.append(gather(W, flat_idx).reshape(B0, th, EMBEDDING_DIM))
    return jnp.concatenate(halves, axis=1)


# speedup vs baseline: 1.0564x; 1.0564x over previous
import functools

import jax
import jax.numpy as jnp
from jax import lax
from jax.experimental import pallas as pl
from jax.experimental.pallas import tpu as pltpu
from jax.experimental.pallas import tpu_sc as plsc

EMBEDDING_DIM = 64


@functools.cache
def _make_flatten(T: int, B0: int):
    n_workers = 32
    bw = B0 // n_workers
    mesh = plsc.VectorSubcoreMesh(core_axis_name="c", subcore_axis_name="s")

    @functools.partial(
        pl.kernel,
        mesh=mesh,
        compiler_params=pltpu.CompilerParams(needs_layout_passes=False),
        out_type=jax.ShapeDtypeStruct((T * B0,), jnp.int32),
        scratch_types=[
            pltpu.VMEM((T, bw), jnp.int32),
            pltpu.VMEM((T * bw,), jnp.int32),
        ],
    )
    def k(xt_hbm, flat_hbm, inb, outb):
        wid = lax.axis_index("s") * 2 + lax.axis_index("c")
        col0 = wid * bw
        pltpu.sync_copy(xt_hbm.at[:, pl.ds(col0, bw)], inb)

        iota = lax.iota(jnp.int32, 16)

        def tbody(t, carry):
            for g in range(bw // 16):
                v = inb[t, pl.ds(g * 16, 16)]
                addr = (iota + g * 16) * T + t
                plsc.store_scatter(outb, [addr], v)
            return carry

        lax.fori_loop(0, T, tbody, 0)
        pltpu.sync_copy(outb, flat_hbm.at[pl.ds(col0 * T, bw * T)])

    return k


@functools.cache
def _make_sc_gather(B: int, D: int, n_workers: int, chunk: int, nbuf: int):
    b_per_w = B // n_workers
    n_chunks = b_per_w // chunk
    n_rounds = (n_chunks + nbuf - 1) // nbuf
    mesh = plsc.VectorSubcoreMesh(core_axis_name="c", subcore_axis_name="s")

    @functools.partial(
        pl.kernel,
        mesh=mesh,
        compiler_params=pltpu.CompilerParams(use_tc_tiling_on_sc=False),
        out_type=jax.ShapeDtypeStruct((B, D), jnp.float32),
        scratch_types=[
            pltpu.VMEM((b_per_w,), jnp.int32),
            pltpu.VMEM((nbuf, chunk, D), jnp.float32),
            pltpu.SemaphoreType.DMA((nbuf,)),
            pltpu.SemaphoreType.DMA((nbuf,)),
        ],
    )
    def k(table_hbm, idx_hbm, out_hbm, idx_v, rows_v, gsem, osem):
        wid = lax.axis_index("s") * 2 + lax.axis_index("c")
        base0 = wid * b_per_w
        pltpu.sync_copy(idx_hbm.at[pl.ds(base0, b_per_w)], idx_v)

        def gather(i, b):
            off = pl.multiple_of(i * chunk, chunk)
            return pltpu.make_async_copy(
                table_hbm.at[idx_v.at[pl.ds(off, chunk)]], rows_v.at[b],
                gsem.at[b]
            )

        def write(i, b):
            off = pl.multiple_of(base0 + i * chunk, chunk)
            return pltpu.make_async_copy(
                rows_v.at[b], out_hbm.at[pl.ds(off, chunk)], osem.at[b]
            )

        for b in range(nbuf):
            gather(b, b).start()

        def round_body(r, carry):
            for b in range(nbuf):
                i = r * nbuf + b

                @pl.when(i < n_chunks)
                def _():
                    gather(i, b).wait()
                    write(i, b).start()
                    nxt = i + nbuf

                    @pl.when(nxt < n_chunks)
                    def _():
                        write(i, b).wait()
                        gather(nxt, b).start()

            return carry

        lax.fori_loop(0, n_rounds, round_body, 0)

        for b in range(nbuf):
            last_i = ((n_chunks - 1 - b) // nbuf) * nbuf + b
            write(last_i, b).wait()

    return k


def kernel(x, W):
    B0, T = x.shape
    B = B0 * T
    xt = x.T.astype(jnp.int32)
    flat_idx = _make_flatten(T, B0)(xt)
    gather = _make_sc_gather(B, EMBEDDING_DIM, 32, 512, 3)
    out = gather(W, flat_idx)
    return out.reshape(B0, T, EMBEDDING_DIM)
